# Initial kernel scaffold; baseline (speedup 1.0000x reference)
import jax
import jax.numpy as jnp
from jax.experimental import pallas as pl


def kernel(x_0, x_1, x_2, adjacency_0, adjacency_1, coadjacency_2, incidence_1, incidence_2, w_l1_00, a_l1_00, ws_l1_01, wt_l1_01, a_l1_01, ws_l1_12, wt_l1_12, a_l1_12, w_l2_00, a_l2_00, w_l2_11, a_l2_11, w_l2_22, a_l2_22, ws_l2_01, wt_l2_01, a_l2_01, ws_l2_12, wt_l2_12, a_l2_12):
    raise NotImplementedError("write your pallas kernel here")


# speedup vs baseline: 4.5180x; 4.5180x over previous
import functools

import jax
import jax.numpy as jnp
from jax import lax
from jax.experimental import pallas as pl
from jax.experimental.pallas import tpu as pltpu
from jax.experimental.pallas import tpu_sc as plsc

D = 128
NEG = 0.2
EPS = 1e-9
BLK = 512
RB = BLK // 128
G = 64
NGRP = BLK // G
TCB = 1000
TILE_TABLE_MAX = 16384


def _round_up(x, m):
    return (x + m - 1) // m * m


@functools.cache
def _sc_attention(n_sc, n_g, dsplit, npass, e_pad):
    nq = 2 * npass
    assert nq * dsplit == D
    acc_rows = _round_up(n_sc, 128)
    ndz = _round_up(n_sc + 16, 512)
    na = _round_up(n_sc + 16, 512)
    nb = _round_up(n_g, 512)
    pa_tile = n_sc <= TILE_TABLE_MAX
    pb_tile = n_g <= TILE_TABLE_MAX
    n_blk = e_pad // 16 // BLK
    rows_t = e_pad // 16 // 128
    stripe = acc_rows // 16
    dchunk = dsplit // 16

    mesh = plsc.VectorSubcoreMesh(core_axis_name="c", subcore_axis_name="s")

    scratch = dict(
        dtab=pltpu.VMEM((ndz,), jnp.float32),
        ebuf=pltpu.VMEM((2, RB, 128), jnp.int32),
        igq=pltpu.VMEM((RB, 128), jnp.int32),
        sbuf=pltpu.VMEM((BLK + 16,), jnp.float32),
        locb=pltpu.VMEM((NGRP, G), jnp.int32),
        rows=pltpu.VMEM((2, G, dsplit), jnp.float32),
        zflat=pltpu.VMEM((512,), jnp.float32),
        pa_v=pltpu.VMEM((RB, 128), jnp.float32),
        pb_v=pltpu.VMEM((RB, 128), jnp.float32),
        acc=pltpu.VMEM_SHARED((acc_rows, dsplit), jnp.float32),
        dshr=pltpu.VMEM_SHARED((ndz,), jnp.float32),
        sem_g0=pltpu.SemaphoreType.DMA,
        sem_g1=pltpu.SemaphoreType.DMA,
        sem_s0=pltpu.SemaphoreType.DMA,
        sem_s1=pltpu.SemaphoreType.DMA,
        sem_t=pltpu.SemaphoreType.DMA,
    )
    scratch["pA"] = (pltpu.VMEM((na,), jnp.float32) if pa_tile
                     else pltpu.VMEM_SHARED((na,), jnp.float32))
    scratch["pB"] = (pltpu.VMEM((nb,), jnp.float32) if pb_tile
                     else pltpu.VMEM_SHARED((nb,), jnp.float32))

    @functools.partial(
        pl.kernel,
        out_type=jax.ShapeDtypeStruct((nq, acc_rows, dsplit), jnp.float32),
        mesh=mesh,
        compiler_params=pltpu.CompilerParams(needs_layout_passes=False,
                                             use_tc_tiling_on_sc=False),
        scratch_types=scratch,
    )
    def kern(edges_hbm, pa_hbm, pb_hbm, s_hbm, out_hbm, dtab, ebuf, igq,
             sbuf, locb, rows, zflat, pa_v, pb_v, acc, dshr,
             sem_g0, sem_g1, sem_s0, sem_s1, sem_t, pA, pB):
        cid = lax.axis_index("c")
        sid = lax.axis_index("s")
        zero16 = jnp.zeros((16,), jnp.float32)
        for k in range(32):
            zflat[pl.ds(k * 16, 16)] = zero16

        @pl.loop(sid, ndz // 512, step=16)
        def _(ch):
            pltpu.sync_copy(zflat, dshr.at[pl.ds(ch * 512, 512)])

        if pa_tile:
            pltpu.sync_copy(pa_hbm, pA)
        else:
            @pl.loop(sid, na // 512, step=16)
            def _(ch):
                pltpu.sync_copy(pa_hbm.at[pl.ds(ch * 512, 512)],
                                pA.at[pl.ds(ch * 512, 512)])
        if pb_tile:
            pltpu.sync_copy(pb_hbm, pB)
        else:
            @pl.loop(sid, nb // 512, step=16)
            def _(ch):
                pltpu.sync_copy(pb_hbm.at[pl.ds(ch * 512, 512)],
                                pB.at[pl.ds(ch * 512, 512)])
        plsc.subcore_barrier()

        def load_edges(i):
            row = sid * rows_t + i * RB
            pltpu.sync_copy(edges_hbm.at[:, pl.ds(row, RB)], ebuf)
            descs = []
            if not pa_tile:
                for r in range(RB):
                    descs.append(pltpu.async_copy(
                        pA.at[ebuf.at[0, r]], pa_v.at[r], sem_t))
            if not pb_tile:
                for r in range(RB):
                    descs.append(pltpu.async_copy(
                        pB.at[ebuf.at[1, r]], pb_v.at[r], sem_t))
            for dd in descs:
                dd.wait()

        def e_chunk(r, l):
            isv = ebuf[0, r, pl.ds(16 * l, 16)]
            igv = ebuf[1, r, pl.ds(16 * l, 16)]
            if pa_tile:
                pa = plsc.load_gather(pA, [isv])
            else:
                pa = pa_v[r, pl.ds(16 * l, 16)]
            if pb_tile:
                pb = plsc.load_gather(pB, [igv])
            else:
                pb = pb_v[r, pl.ds(16 * l, 16)]
            z = pa + pb
            e = jnp.where(z >= 0, z, NEG * z)
            return isv, igv, e

        @pl.loop(0, n_blk)
        def _(i):
            load_edges(i)
            for r in range(RB):
                for l in range(8):
                    isv, _, e = e_chunk(r, l)
                    valid = isv < n_sc
                    sbuf[pl.ds(r * 128 + 16 * l, 16)] = jnp.where(valid, e, 0.0)
            descs = []
            for r in range(RB):
                descs.append(pltpu.async_copy(
                    sbuf.at[pl.ds(r * 128, 128)], dshr.at[ebuf.at[0, r]],
                    sem_s0, add=True))
            for dd in descs:
                dd.wait()

        plsc.subcore_barrier()
        pltpu.sync_copy(dshr, dtab)

        for p in range(npass):
            plsc.subcore_barrier()
            for j in range(16):
                for k in range(dchunk):
                    rows[0, j, pl.ds(16 * k, 16)] = zero16

            @pl.loop(sid, acc_rows // 16, step=16)
            def _(ch):
                pltpu.sync_copy(rows.at[0, pl.ds(0, 16)],
                                acc.at[pl.ds(ch * 16, 16)])

            plsc.subcore_barrier()
            qoff = (2 * p + cid) * n_g

            @pl.loop(0, n_blk)
            def _(i):
                load_edges(i)
                for r in range(RB):
                    for l in range(8):
                        isv, igv, e = e_chunk(r, l)
                        dd = plsc.load_gather(dtab, [isv])
                        cc = e / (dd + EPS)
                        valid = isv < n_sc
                        cc = jnp.where(valid, cc, 0.0)
                        loc = jnp.where(valid, isv, 0)
                        sbuf[pl.ds(r * 128 + 16 * l, 16)] = cc
                        gi = (r * 128 + 16 * l) // G
                        locb[gi, pl.ds((16 * l) % G, 16)] = loc
                        igq[r, pl.ds(16 * l, 16)] = igv + qoff
                gd = [None, None]
                sd = [None, None]
                gsem = [sem_g0, sem_g1]
                ssem = [sem_s0, sem_s1]
                for g in range(NGRP + 1):
                    if g < NGRP:
                        b = g % 2
                        if sd[b] is not None:
                            sd[b].wait()
                            sd[b] = None
                        idxv = igq.at[(g * G) // 128, pl.ds((g * G) % 128, G)]
                        gd[b] = pltpu.async_copy(s_hbm.at[idxv], rows.at[b],
                                                 gsem[b])
                    if g >= 1:
                        gp = g - 1
                        b = gp % 2
                        gd[b].wait()

                        @pl.loop(0, G)
                        def _(j):
                            cv = sbuf[pl.ds(gp * G + j, 16)]
                            cb = jnp.full((16,), cv[0], jnp.float32)
                            for k in range(dchunk):
                                rows[b, j, pl.ds(16 * k, 16)] = (
                                    rows[b, j, pl.ds(16 * k, 16)] * cb)

                        sd[b] = pltpu.async_copy(
                            rows.at[b], acc.at[locb.at[gp]], ssem[b], add=True)
                for b in range(2):
                    if sd[b] is not None:
                        sd[b].wait()

            plsc.subcore_barrier()
            pltpu.sync_copy(acc.at[pl.ds(sid * stripe, stripe)],
                            out_hbm.at[2 * p + cid,
                                       pl.ds(sid * stripe, stripe)])

    return kern


def _tc_stage(sources, pairs, n, return_x=False):
    grid_n = n // TCB
    in_specs = []
    args = []
    for arr, kind in sources:
        if kind == "plain":
            in_specs.append(pl.BlockSpec((TCB, D), lambda i: (i, 0)))
        else:
            nq_s = arr.shape[0]
            ds_s = arr.shape[2]
            in_specs.append(
                pl.BlockSpec((nq_s, TCB, ds_s), lambda i: (0, i, 0)))
        args.append(arr)
    for w, a in pairs:
        in_specs.append(pl.BlockSpec((D, D), lambda i: (0, 0)))
        in_specs.append(pl.BlockSpec((D, D), lambda i: (0, 0)))
        args += [w, a]

    n_src = len(sources)
    n_out = (1 if return_x else 0) + 2 * len(pairs)
    out_specs = [pl.BlockSpec((TCB, D), lambda i: (i, 0))] * n_out
    out_shape = [jax.ShapeDtypeStruct((n, D), jnp.float32)] * n_out

    def body(*refs):
        ins = refs[:n_src + 2 * len(pairs)]
        outs = refs[n_src + 2 * len(pairs):]
        x = None
        for j, (_, kind) in enumerate(sources):
            v = ins[j][...]
            if kind == "dsplit":
                v = jnp.concatenate([v[q] for q in range(v.shape[0])], axis=1)
            x = v if x is None else x + v
        oi = 0
        if return_x:
            outs[0][...] = x
            oi = 1
        for k in range(len(pairs)):
            w = ins[n_src + 2 * k][...]
            a = ins[n_src + 2 * k + 1][...]
            msg = jnp.dot(x, w, preferred_element_type=jnp.float32)
            outs[oi][...] = msg
            outs[oi + 1][...] = jnp.dot(msg, a,
                                        preferred_element_type=jnp.float32)
            oi += 2

    outs = pl.pallas_call(
        body,
        grid=(grid_n,),
        in_specs=in_specs,
        out_specs=out_specs,
        out_shape=out_shape,
    )(*args)
    return outs


def _att_mat(att):
    a = jnp.zeros((D, D), jnp.float32)
    a = a.at[:, 0].set(att[:D])
    a = a.at[:, 1].set(att[D:])
    return a


def _att_col(col):
    a = jnp.zeros((D, D), jnp.float32)
    return a.at[:, 0].set(col)


def _pack_edges(seg, oth, pad_val):
    e = seg.shape[0]
    e_pad = _round_up(e, 8192)
    s = jnp.pad(seg, (0, e_pad - e), constant_values=pad_val)
    g = jnp.pad(oth, (0, e_pad - e), constant_values=0)
    return jnp.stack([s, g]).reshape(2, e_pad // 128, 128), e_pad


def _pad_table(v, m):
    return jnp.pad(v, (0, _round_up(v.shape[0] + m, 512) - v.shape[0]))


def _split_cols(s, nq):
    n = s.shape[0]
    w = D // nq
    return s.reshape(n, nq, w).transpose(1, 0, 2).reshape(nq * n, w)


def _sc_block(edges, e_pad, pa, pb, s, n_sc, n_g):
    dsplit, npass = (64, 1) if n_sc <= TILE_TABLE_MAX else (32, 2)
    fn = _sc_attention(n_sc, n_g, dsplit, npass, e_pad)
    return fn(edges, _pad_table(pa, 16), _pad_table(pb, 0),
              _split_cols(s, 2 * npass))


def kernel(x_0, x_1, x_2, adjacency_0, adjacency_1, coadjacency_2,
           incidence_1, incidence_2, w_l1_00, a_l1_00, ws_l1_01, wt_l1_01,
           a_l1_01, ws_l1_12, wt_l1_12, a_l1_12, w_l2_00, a_l2_00, w_l2_11,
           a_l2_11, w_l2_22, a_l2_22, ws_l2_01, wt_l2_01, a_l2_01, ws_l2_12,
           wt_l2_12, a_l2_12):
    n0, n1, n2 = x_0.shape[0], x_1.shape[0], x_2.shape[0]

    adj0, ep_adj0 = _pack_edges(adjacency_0[0], adjacency_0[1], n0)
    adj1, ep_adj1 = _pack_edges(adjacency_1[0], adjacency_1[1], n1)
    cadj2, ep_cadj2 = _pack_edges(coadjacency_2[0], coadjacency_2[1], n2)
    inc1_t, ep_inc1t = _pack_edges(incidence_1[0], incidence_1[1], n1)
    inc1_s, ep_inc1s = _pack_edges(incidence_1[1], incidence_1[0], n0)
    inc2_t, ep_inc2t = _pack_edges(incidence_2[0], incidence_2[1], n2)
    inc2_s, ep_inc2s = _pack_edges(incidence_2[1], incidence_2[0], n1)

    msg00, pq00, s01, sp01m = _tc_stage(
        [(x_0, "plain")],
        [(w_l1_00, _att_mat(a_l1_00)), (ws_l1_01, _att_col(a_l1_01[:D]))], n0)
    t01, tq01m, s12, sp12m = _tc_stage(
        [(x_1, "plain")],
        [(wt_l1_01, _att_col(a_l1_01[D:])), (ws_l1_12, _att_col(a_l1_12[:D]))],
        n1)
    t12, tq12m = _tc_stage(
        [(x_2, "plain")], [(wt_l1_12, _att_col(a_l1_12[D:]))], n2)

    slab00 = _sc_block(adj0, ep_adj0, pq00[:, 0], pq00[:, 1], msg00, n0, n0)
    slab01t = _sc_block(inc1_t, ep_inc1t, tq01m[:, 0], sp01m[:, 0], s01,
                        n1, n0)
    slab01s = _sc_block(inc1_s, ep_inc1s, sp01m[:, 0], tq01m[:, 0], t01,
                        n0, n1)
    slab12t = _sc_block(inc2_t, ep_inc2t, tq12m[:, 0], sp12m[:, 0], s12,
                        n2, n1)
    slab12s = _sc_block(inc2_s, ep_inc2s, sp12m[:, 0], tq12m[:, 0], t12,
                        n1, n2)

    m00, pq00b, s01b, sp01bm = _tc_stage(
        [(slab00, "dsplit"), (slab01s, "dsplit")],
        [(w_l2_00, _att_mat(a_l2_00)), (ws_l2_01, _att_col(a_l2_01[:D]))], n0)
    m11, pq11b, t01b, tq01bm, s12b, sp12bm = _tc_stage(
        [(slab01t, "dsplit"), (slab12s, "dsplit")],
        [(w_l2_11, _att_mat(a_l2_11)), (wt_l2_01, _att_col(a_l2_01[D:])),
         (ws_l2_12, _att_col(a_l2_12[:D]))], n1)
    m22, pq22b, t12b, tq12bm = _tc_stage(
        [(slab12t, "dsplit")],
        [(w_l2_22, _att_mat(a_l2_22)), (wt_l2_12, _att_col(a_l2_12[D:]))], n2)

    slaby00 = _sc_block(adj0, ep_adj0, pq00b[:, 0], pq00b[:, 1], m00, n0, n0)
    slaby11 = _sc_block(adj1, ep_adj1, pq11b[:, 0], pq11b[:, 1], m11, n1, n1)
    slaby22 = _sc_block(cadj2, ep_cadj2, pq22b[:, 0], pq22b[:, 1], m22,
                        n2, n2)
    slaby01 = _sc_block(inc1_t, ep_inc1t, tq01bm[:, 0], sp01bm[:, 0], s01b,
                        n1, n0)
    slaby12 = _sc_block(inc2_t, ep_inc2t, tq12bm[:, 0], sp12bm[:, 0], s12b,
                        n2, n1)

    (x0_out,) = _tc_stage([(slaby00, "dsplit")], [], n0, return_x=True)
    (x1_out,) = _tc_stage([(slaby01, "dsplit"), (slaby11, "dsplit")], [], n1,
                          return_x=True)
    (x2_out,) = _tc_stage([(slaby12, "dsplit"), (slaby22, "dsplit")], [], n2,
                          return_x=True)
    return (x0_out, x1_out, x2_out)


# speedup vs baseline: 4.5783x; 1.0133x over previous
import functools

import jax
import jax.numpy as jnp
from jax import lax
from jax.experimental import pallas as pl
from jax.experimental.pallas import tpu as pltpu
from jax.experimental.pallas import tpu_sc as plsc

D = 128
NEG = 0.2
EPS = 1e-9
BLK = 512
RB = BLK // 128
G = 128
NGRP = BLK // G
TCB = 1000
TILE_TABLE_MAX = 16384


def _round_up(x, m):
    return (x + m - 1) // m * m


@functools.cache
def _sc_attention(n_sc, n_g, dsplit, npass, e_pad):
    nq = 2 * npass
    assert nq * dsplit == D
    acc_rows = _round_up(n_sc, 128)
    ndz = _round_up(n_sc + 16, 512)
    na = _round_up(n_sc + 16, 512)
    nb = _round_up(n_g, 512)
    pa_tile = n_sc <= TILE_TABLE_MAX
    pb_tile = n_g <= TILE_TABLE_MAX
    n_blk = e_pad // 16 // BLK
    rows_t = e_pad // 16 // 128
    stripe = acc_rows // 16
    dchunk = dsplit // 16

    mesh = plsc.VectorSubcoreMesh(core_axis_name="c", subcore_axis_name="s")

    scratch = dict(
        dtab=pltpu.VMEM((ndz,), jnp.float32),
        ebuf=pltpu.VMEM((2, RB, 128), jnp.int32),
        igq=pltpu.VMEM((RB, 128), jnp.int32),
        sbuf=pltpu.VMEM((BLK + 16,), jnp.float32),
        locb=pltpu.VMEM((NGRP, G), jnp.int32),
        rows=pltpu.VMEM((2, G, dsplit), jnp.float32),
        zflat=pltpu.VMEM((512,), jnp.float32),
        pa_v=pltpu.VMEM((RB, 128), jnp.float32),
        pb_v=pltpu.VMEM((RB, 128), jnp.float32),
        acc=pltpu.VMEM_SHARED((acc_rows, dsplit), jnp.float32),
        dshr=pltpu.VMEM_SHARED((ndz,), jnp.float32),
        sem_g0=pltpu.SemaphoreType.DMA,
        sem_g1=pltpu.SemaphoreType.DMA,
        sem_s0=pltpu.SemaphoreType.DMA,
        sem_s1=pltpu.SemaphoreType.DMA,
        sem_t=pltpu.SemaphoreType.DMA,
    )
    scratch["pA"] = (pltpu.VMEM((na,), jnp.float32) if pa_tile
                     else pltpu.VMEM_SHARED((na,), jnp.float32))
    scratch["pB"] = (pltpu.VMEM((nb,), jnp.float32) if pb_tile
                     else pltpu.VMEM_SHARED((nb,), jnp.float32))

    @functools.partial(
        pl.kernel,
        out_type=jax.ShapeDtypeStruct((nq, acc_rows, dsplit), jnp.float32),
        mesh=mesh,
        compiler_params=pltpu.CompilerParams(needs_layout_passes=False,
                                             use_tc_tiling_on_sc=False),
        scratch_types=scratch,
    )
    def kern(edges_hbm, pa_hbm, pb_hbm, s_hbm, out_hbm, dtab, ebuf, igq,
             sbuf, locb, rows, zflat, pa_v, pb_v, acc, dshr,
             sem_g0, sem_g1, sem_s0, sem_s1, sem_t, pA, pB):
        cid = lax.axis_index("c")
        sid = lax.axis_index("s")
        zero16 = jnp.zeros((16,), jnp.float32)
        for k in range(32):
            zflat[pl.ds(k * 16, 16)] = zero16

        @pl.loop(sid, ndz // 512, step=16)
        def _(ch):
            pltpu.sync_copy(zflat, dshr.at[pl.ds(ch * 512, 512)])

        if pa_tile:
            pltpu.sync_copy(pa_hbm, pA)
        else:
            @pl.loop(sid, na // 512, step=16)
            def _(ch):
                pltpu.sync_copy(pa_hbm.at[pl.ds(ch * 512, 512)],
                                pA.at[pl.ds(ch * 512, 512)])
        if pb_tile:
            pltpu.sync_copy(pb_hbm, pB)
        else:
            @pl.loop(sid, nb // 512, step=16)
            def _(ch):
                pltpu.sync_copy(pb_hbm.at[pl.ds(ch * 512, 512)],
                                pB.at[pl.ds(ch * 512, 512)])
        plsc.subcore_barrier()

        def load_edges(i):
            row = sid * rows_t + i * RB
            pltpu.sync_copy(edges_hbm.at[:, pl.ds(row, RB)], ebuf)
            descs = []
            if not pa_tile:
                for r in range(RB):
                    descs.append(pltpu.async_copy(
                        pA.at[ebuf.at[0, r]], pa_v.at[r], sem_t))
            if not pb_tile:
                for r in range(RB):
                    descs.append(pltpu.async_copy(
                        pB.at[ebuf.at[1, r]], pb_v.at[r], sem_t))
            for dd in descs:
                dd.wait()

        def e_chunk(r, l):
            isv = ebuf[0, r, pl.ds(16 * l, 16)]
            igv = ebuf[1, r, pl.ds(16 * l, 16)]
            if pa_tile:
                pa = plsc.load_gather(pA, [isv])
            else:
                pa = pa_v[r, pl.ds(16 * l, 16)]
            if pb_tile:
                pb = plsc.load_gather(pB, [igv])
            else:
                pb = pb_v[r, pl.ds(16 * l, 16)]
            z = pa + pb
            e = jnp.where(z >= 0, z, NEG * z)
            return isv, igv, e

        @pl.loop(0, n_blk)
        def _(i):
            load_edges(i)
            for r in range(RB):
                for l in range(8):
                    isv, _, e = e_chunk(r, l)
                    valid = isv < n_sc
                    sbuf[pl.ds(r * 128 + 16 * l, 16)] = jnp.where(valid, e, 0.0)
            descs = []
            for r in range(RB):
                descs.append(pltpu.async_copy(
                    sbuf.at[pl.ds(r * 128, 128)], dshr.at[ebuf.at[0, r]],
                    sem_s0, add=True))
            for dd in descs:
                dd.wait()

        plsc.subcore_barrier()
        pltpu.sync_copy(dshr, dtab)

        for p in range(npass):
            plsc.subcore_barrier()
            for j in range(16):
                for k in range(dchunk):
                    rows[0, j, pl.ds(16 * k, 16)] = zero16

            @pl.loop(sid, acc_rows // 16, step=16)
            def _(ch):
                pltpu.sync_copy(rows.at[0, pl.ds(0, 16)],
                                acc.at[pl.ds(ch * 16, 16)])

            plsc.subcore_barrier()
            qoff = (2 * p + cid) * n_g

            @pl.loop(0, n_blk)
            def _(i):
                load_edges(i)
                for r in range(RB):
                    for l in range(8):
                        isv, igv, e = e_chunk(r, l)
                        dd = plsc.load_gather(dtab, [isv])
                        cc = e / (dd + EPS)
                        valid = isv < n_sc
                        cc = jnp.where(valid, cc, 0.0)
                        loc = jnp.where(valid, isv, 0)
                        sbuf[pl.ds(r * 128 + 16 * l, 16)] = cc
                        gi = (r * 128 + 16 * l) // G
                        locb[gi, pl.ds((16 * l) % G, 16)] = loc
                        igq[r, pl.ds(16 * l, 16)] = igv + qoff
                gd = [None, None]
                sd = [None, None]
                gsem = [sem_g0, sem_g1]
                ssem = [sem_s0, sem_s1]
                for g in range(NGRP + 1):
                    if g < NGRP:
                        b = g % 2
                        if sd[b] is not None:
                            sd[b].wait()
                            sd[b] = None
                        idxv = igq.at[(g * G) // 128, pl.ds((g * G) % 128, G)]
                        gd[b] = pltpu.async_copy(s_hbm.at[idxv], rows.at[b],
                                                 gsem[b])
                    if g >= 1:
                        gp = g - 1
                        b = gp % 2
                        gd[b].wait()

                        @pl.loop(0, G)
                        def _(j):
                            cv = sbuf[pl.ds(gp * G + j, 16)]
                            cb = jnp.full((16,), cv[0], jnp.float32)
                            for k in range(dchunk):
                                rows[b, j, pl.ds(16 * k, 16)] = (
                                    rows[b, j, pl.ds(16 * k, 16)] * cb)

                        sd[b] = pltpu.async_copy(
                            rows.at[b], acc.at[locb.at[gp]], ssem[b], add=True)
                for b in range(2):
                    if sd[b] is not None:
                        sd[b].wait()

            plsc.subcore_barrier()
            pltpu.sync_copy(acc.at[pl.ds(sid * stripe, stripe)],
                            out_hbm.at[2 * p + cid,
                                       pl.ds(sid * stripe, stripe)])

    return kern


def _tc_stage(sources, pairs, n, return_x=False):
    grid_n = n // TCB
    in_specs = []
    args = []
    for arr, kind in sources:
        if kind == "plain":
            in_specs.append(pl.BlockSpec((TCB, D), lambda i: (i, 0)))
        else:
            nq_s = arr.shape[0]
            ds_s = arr.shape[2]
            in_specs.append(
                pl.BlockSpec((nq_s, TCB, ds_s), lambda i: (0, i, 0)))
        args.append(arr)
    for w, a in pairs:
        in_specs.append(pl.BlockSpec((D, D), lambda i: (0, 0)))
        in_specs.append(pl.BlockSpec((D, D), lambda i: (0, 0)))
        args += [w, a]

    n_src = len(sources)
    n_out = (1 if return_x else 0) + 2 * len(pairs)
    out_specs = [pl.BlockSpec((TCB, D), lambda i: (i, 0))] * n_out
    out_shape = [jax.ShapeDtypeStruct((n, D), jnp.float32)] * n_out

    def body(*refs):
        ins = refs[:n_src + 2 * len(pairs)]
        outs = refs[n_src + 2 * len(pairs):]
        x = None
        for j, (_, kind) in enumerate(sources):
            v = ins[j][...]
            if kind == "dsplit":
                v = jnp.concatenate([v[q] for q in range(v.shape[0])], axis=1)
            x = v if x is None else x + v
        oi = 0
        if return_x:
            outs[0][...] = x
            oi = 1
        for k in range(len(pairs)):
            w = ins[n_src + 2 * k][...]
            a = ins[n_src + 2 * k + 1][...]
            msg = jnp.dot(x, w, preferred_element_type=jnp.float32)
            outs[oi][...] = msg
            outs[oi + 1][...] = jnp.dot(msg, a,
                                        preferred_element_type=jnp.float32)
            oi += 2

    outs = pl.pallas_call(
        body,
        grid=(grid_n,),
        in_specs=in_specs,
        out_specs=out_specs,
        out_shape=out_shape,
    )(*args)
    return outs


def _att_mat(att):
    a = jnp.zeros((D, D), jnp.float32)
    a = a.at[:, 0].set(att[:D])
    a = a.at[:, 1].set(att[D:])
    return a


def _att_col(col):
    a = jnp.zeros((D, D), jnp.float32)
    return a.at[:, 0].set(col)


def _pack_edges(seg, oth, pad_val):
    e = seg.shape[0]
    e_pad = _round_up(e, 8192)
    s = jnp.pad(seg, (0, e_pad - e), constant_values=pad_val)
    g = jnp.pad(oth, (0, e_pad - e), constant_values=0)
    return jnp.stack([s, g]).reshape(2, e_pad // 128, 128), e_pad


def _pad_table(v, m):
    return jnp.pad(v, (0, _round_up(v.shape[0] + m, 512) - v.shape[0]))


def _split_cols(s, nq):
    n = s.shape[0]
    w = D // nq
    return s.reshape(n, nq, w).transpose(1, 0, 2).reshape(nq * n, w)


def _sc_block(edges, e_pad, pa, pb, s, n_sc, n_g):
    dsplit, npass = (64, 1) if n_sc <= TILE_TABLE_MAX else (32, 2)
    fn = _sc_attention(n_sc, n_g, dsplit, npass, e_pad)
    return fn(edges, _pad_table(pa, 16), _pad_table(pb, 0),
              _split_cols(s, 2 * npass))


def kernel(x_0, x_1, x_2, adjacency_0, adjacency_1, coadjacency_2,
           incidence_1, incidence_2, w_l1_00, a_l1_00, ws_l1_01, wt_l1_01,
           a_l1_01, ws_l1_12, wt_l1_12, a_l1_12, w_l2_00, a_l2_00, w_l2_11,
           a_l2_11, w_l2_22, a_l2_22, ws_l2_01, wt_l2_01, a_l2_01, ws_l2_12,
           wt_l2_12, a_l2_12):
    n0, n1, n2 = x_0.shape[0], x_1.shape[0], x_2.shape[0]

    adj0, ep_adj0 = _pack_edges(adjacency_0[0], adjacency_0[1], n0)
    adj1, ep_adj1 = _pack_edges(adjacency_1[0], adjacency_1[1], n1)
    cadj2, ep_cadj2 = _pack_edges(coadjacency_2[0], coadjacency_2[1], n2)
    inc1_t, ep_inc1t = _pack_edges(incidence_1[0], incidence_1[1], n1)
    inc1_s, ep_inc1s = _pack_edges(incidence_1[1], incidence_1[0], n0)
    inc2_t, ep_inc2t = _pack_edges(incidence_2[0], incidence_2[1], n2)
    inc2_s, ep_inc2s = _pack_edges(incidence_2[1], incidence_2[0], n1)

    msg00, pq00, s01, sp01m = _tc_stage(
        [(x_0, "plain")],
        [(w_l1_00, _att_mat(a_l1_00)), (ws_l1_01, _att_col(a_l1_01[:D]))], n0)
    t01, tq01m, s12, sp12m = _tc_stage(
        [(x_1, "plain")],
        [(wt_l1_01, _att_col(a_l1_01[D:])), (ws_l1_12, _att_col(a_l1_12[:D]))],
        n1)
    t12, tq12m = _tc_stage(
        [(x_2, "plain")], [(wt_l1_12, _att_col(a_l1_12[D:]))], n2)

    slab00 = _sc_block(adj0, ep_adj0, pq00[:, 0], pq00[:, 1], msg00, n0, n0)
    slab01t = _sc_block(inc1_t, ep_inc1t, tq01m[:, 0], sp01m[:, 0], s01,
                        n1, n0)
    slab01s = _sc_block(inc1_s, ep_inc1s, sp01m[:, 0], tq01m[:, 0], t01,
                        n0, n1)
    slab12t = _sc_block(inc2_t, ep_inc2t, tq12m[:, 0], sp12m[:, 0], s12,
                        n2, n1)
    slab12s = _sc_block(inc2_s, ep_inc2s, sp12m[:, 0], tq12m[:, 0], t12,
                        n1, n2)

    m00, pq00b, s01b, sp01bm = _tc_stage(
        [(slab00, "dsplit"), (slab01s, "dsplit")],
        [(w_l2_00, _att_mat(a_l2_00)), (ws_l2_01, _att_col(a_l2_01[:D]))], n0)
    m11, pq11b, t01b, tq01bm, s12b, sp12bm = _tc_stage(
        [(slab01t, "dsplit"), (slab12s, "dsplit")],
        [(w_l2_11, _att_mat(a_l2_11)), (wt_l2_01, _att_col(a_l2_01[D:])),
         (ws_l2_12, _att_col(a_l2_12[:D]))], n1)
    m22, pq22b, t12b, tq12bm = _tc_stage(
        [(slab12t, "dsplit")],
        [(w_l2_22, _att_mat(a_l2_22)), (wt_l2_12, _att_col(a_l2_12[D:]))], n2)

    slaby00 = _sc_block(adj0, ep_adj0, pq00b[:, 0], pq00b[:, 1], m00, n0, n0)
    slaby11 = _sc_block(adj1, ep_adj1, pq11b[:, 0], pq11b[:, 1], m11, n1, n1)
    slaby22 = _sc_block(cadj2, ep_cadj2, pq22b[:, 0], pq22b[:, 1], m22,
                        n2, n2)
    slaby01 = _sc_block(inc1_t, ep_inc1t, tq01bm[:, 0], sp01bm[:, 0], s01b,
                        n1, n0)
    slaby12 = _sc_block(inc2_t, ep_inc2t, tq12bm[:, 0], sp12bm[:, 0], s12b,
                        n2, n1)

    (x0_out,) = _tc_stage([(slaby00, "dsplit")], [], n0, return_x=True)
    (x1_out,) = _tc_stage([(slaby01, "dsplit"), (slaby11, "dsplit")], [], n1,
                          return_x=True)
    (x2_out,) = _tc_stage([(slaby12, "dsplit"), (slaby22, "dsplit")], [], n2,
                          return_x=True)
    return (x0_out, x1_out, x2_out)
